# Initial kernel scaffold; baseline (speedup 1.0000x reference)
import functools

import jax
import jax.numpy as jnp
from jax import lax
from jax.experimental import pallas as pl
from jax.experimental.pallas import tpu as pltpu
from jax.experimental.pallas import tpu_sc as plsc

N = 10000
D = 128
E = 320000

NC = 2
NS = 16
NW = NC * NS
K = 128
CH = 79
EPW = K * CH
EPAD = EPW * NW
NPAD = 10016
ZROWS = NPAD // NS
OROWS = N // NS
DW = 16

_mesh = plsc.VectorSubcoreMesh(core_axis_name="c", subcore_axis_name="s")


def _deg_body(dst_hbm, out_hbm, dst_v, buf_v, acc_sh):
    c = lax.axis_index("c")
    s = lax.axis_index("s")
    wid = s * NC + c

    def _zrow(i, carry):
        buf_v[i] = jnp.zeros((DW,), jnp.float32)
        return carry

    lax.fori_loop(0, K, _zrow, 0)
    for m in range(ZROWS // K):
        pltpu.sync_copy(buf_v, acc_sh.at[pl.ds(s * ZROWS + m * K, K)])
    rem = ZROWS % K
    if rem:
        pltpu.sync_copy(buf_v.at[pl.ds(0, rem)],
                        acc_sh.at[pl.ds(s * ZROWS + (ZROWS // K) * K, rem)])

    def _orow(i, carry):
        buf_v[i] = jnp.ones((DW,), jnp.float32)
        return carry

    lax.fori_loop(0, K, _orow, 0)
    plsc.subcore_barrier()

    def _body(j, carry):
        off = wid * EPW + j * K
        pltpu.sync_copy(dst_hbm.at[pl.ds(off, K)], dst_v)
        pltpu.sync_copy(buf_v, acc_sh.at[dst_v], add=True)
        return carry

    lax.fori_loop(0, CH, _body, 0)
    plsc.subcore_barrier()

    pltpu.sync_copy(acc_sh.at[pl.ds(s * OROWS, OROWS)],
                    out_hbm.at[c, pl.ds(s * OROWS, OROWS)])


_deg_kernel = pl.kernel(
    _deg_body,
    out_type=jax.ShapeDtypeStruct((NC, N, DW), jnp.float32),
    mesh=_mesh,
    scratch_types=[
        pltpu.VMEM((K,), jnp.int32),
        pltpu.VMEM((K, DW), jnp.float32),
        pltpu.VMEM_SHARED((NPAD, DW), jnp.float32),
    ],
)


def _edge_body(g_hbm, src_hbm, dst_hbm, out_hbm, src_v, dst_v, rows_v, acc_sh,
               sem):
    c = lax.axis_index("c")
    s = lax.axis_index("s")
    wid = s * NC + c

    def _zrow(i, carry):
        for j in range(D // 16):
            rows_v[i, pl.ds(j * 16, 16)] = jnp.zeros((16,), jnp.float32)
        return carry

    lax.fori_loop(0, K, _zrow, 0)
    for m in range(ZROWS // K):
        pltpu.sync_copy(rows_v, acc_sh.at[pl.ds(s * ZROWS + m * K, K)])
    rem = ZROWS % K
    if rem:
        pltpu.sync_copy(rows_v.at[pl.ds(0, rem)],
                        acc_sh.at[pl.ds(s * ZROWS + (ZROWS // K) * K, rem)])
    plsc.subcore_barrier()

    def _body(j, carry):
        off = wid * EPW + j * K
        pltpu.sync_copy(src_hbm.at[pl.ds(off, K)], src_v)
        pltpu.sync_copy(dst_hbm.at[pl.ds(off, K)], dst_v)
        pltpu.async_copy(g_hbm.at[src_v], rows_v, sem).wait()
        pltpu.sync_copy(rows_v, acc_sh.at[dst_v], add=True)
        return carry

    lax.fori_loop(0, CH, _body, 0)
    plsc.subcore_barrier()

    pltpu.sync_copy(acc_sh.at[pl.ds(s * OROWS, OROWS)],
                    out_hbm.at[c, pl.ds(s * OROWS, OROWS)])


_edge_kernel = pl.kernel(
    _edge_body,
    out_type=jax.ShapeDtypeStruct((NC, N, D), jnp.float32),
    mesh=_mesh,
    scratch_types=[
        pltpu.VMEM((K,), jnp.int32),
        pltpu.VMEM((K,), jnp.int32),
        pltpu.VMEM((K, D), jnp.float32),
        pltpu.VMEM_SHARED((NPAD, D), jnp.float32),
        pltpu.SemaphoreType.DMA,
    ],
)


R = 1000


def _tc1_body(x_ref, w1_ref, p_ref, h_ref, g_ref, dinv_ref):
    p = p_ref[...]
    deg = 1.0 + p[0, :, 0:1] + p[1, :, 0:1]
    dinv = lax.rsqrt(deg)
    h = jnp.dot(x_ref[...], w1_ref[...], preferred_element_type=jnp.float32)
    h_ref[...] = h
    g_ref[...] = h * dinv
    dinv_ref[...] = dinv


def _tc1(x, W1, p):
    return pl.pallas_call(
        _tc1_body,
        grid=(N // R,),
        in_specs=[
            pl.BlockSpec((R, D), lambda i: (i, 0)),
            pl.BlockSpec((D, D), lambda i: (0, 0)),
            pl.BlockSpec((NC, R, DW), lambda i: (0, i, 0)),
        ],
        out_specs=[
            pl.BlockSpec((R, D), lambda i: (i, 0)),
            pl.BlockSpec((R, D), lambda i: (i, 0)),
            pl.BlockSpec((R, 1), lambda i: (i, 0)),
        ],
        out_shape=[
            jax.ShapeDtypeStruct((N, D), jnp.float32),
            jax.ShapeDtypeStruct((N, D), jnp.float32),
            jax.ShapeDtypeStruct((N, 1), jnp.float32),
        ],
    )(x, W1, p)


def _tc2_body(acc_ref, h1_ref, dinv_ref, b1_ref, w2_ref, h2_ref, g2_ref):
    dinv = dinv_ref[...]
    a = acc_ref[0] + acc_ref[1]
    z = jnp.maximum(a * dinv + h1_ref[...] * (dinv * dinv) + b1_ref[...], 0.0)
    h2 = jnp.dot(z, w2_ref[...], preferred_element_type=jnp.float32)
    h2_ref[...] = h2
    g2_ref[...] = h2 * dinv


def _tc2(acc, h1, dinv, b1, W2):
    return pl.pallas_call(
        _tc2_body,
        grid=(N // R,),
        in_specs=[
            pl.BlockSpec((NC, R, D), lambda i: (0, i, 0)),
            pl.BlockSpec((R, D), lambda i: (i, 0)),
            pl.BlockSpec((R, 1), lambda i: (i, 0)),
            pl.BlockSpec((1, D), lambda i: (0, 0)),
            pl.BlockSpec((D, D), lambda i: (0, 0)),
        ],
        out_specs=[
            pl.BlockSpec((R, D), lambda i: (i, 0)),
            pl.BlockSpec((R, D), lambda i: (i, 0)),
        ],
        out_shape=[
            jax.ShapeDtypeStruct((N, D), jnp.float32),
            jax.ShapeDtypeStruct((N, D), jnp.float32),
        ],
    )(acc, h1, dinv, b1, W2)


def _tc3_body(acc_ref, h2_ref, dinv_ref, b2_ref, fw1_ref, fb1_ref, fw2_ref,
              fb2_ref, out_ref):
    dinv = dinv_ref[...]
    a = acc_ref[0] + acc_ref[1]
    z = jnp.maximum(a * dinv + h2_ref[...] * (dinv * dinv) + b2_ref[...], 0.0)
    t = jnp.maximum(
        jnp.dot(z, fw1_ref[...], preferred_element_type=jnp.float32)
        + fb1_ref[...], 0.0)
    out_ref[...] = (
        jnp.dot(t, fw2_ref[...], preferred_element_type=jnp.float32)
        + fb2_ref[...])


def _tc3(acc, h2, dinv, b2, fcW1, fcb1, fcW2, fcb2):
    return pl.pallas_call(
        _tc3_body,
        grid=(N // R,),
        in_specs=[
            pl.BlockSpec((NC, R, D), lambda i: (0, i, 0)),
            pl.BlockSpec((R, D), lambda i: (i, 0)),
            pl.BlockSpec((R, 1), lambda i: (i, 0)),
            pl.BlockSpec((1, D), lambda i: (0, 0)),
            pl.BlockSpec((D, D), lambda i: (0, 0)),
            pl.BlockSpec((1, D), lambda i: (0, 0)),
            pl.BlockSpec((D, D), lambda i: (0, 0)),
            pl.BlockSpec((1, D), lambda i: (0, 0)),
        ],
        out_specs=pl.BlockSpec((R, D), lambda i: (i, 0)),
        out_shape=jax.ShapeDtypeStruct((N, D), jnp.float32),
    )(acc, h2, dinv, b2, fcW1, fcb1, fcW2, fcb2)


def kernel(x, edge_index, W1, b1, W2, b2, fcW1, fcb1, fcW2, fcb2):
    src = edge_index[0]
    dst = edge_index[1]
    pad = EPAD - E
    src_p = jnp.concatenate([src, jnp.zeros((pad,), jnp.int32)])
    dst_p = jnp.concatenate([dst, jnp.full((pad,), N, jnp.int32)])

    p = _deg_kernel(dst_p)
    h1, g1, dinv = _tc1(x, W1, p)
    acc1 = _edge_kernel(g1, src_p, dst_p)
    h2, g2 = _tc2(acc1, h1, dinv, b1.reshape(1, D), W2)
    acc2 = _edge_kernel(g2, src_p, dst_p)
    return _tc3(acc2, h2, dinv, b2.reshape(1, D), fcW1, fcb1.reshape(1, D),
                fcW2, fcb2.reshape(1, D))


# speedup vs baseline: 10.9260x; 10.9260x over previous
import functools

import jax
import jax.numpy as jnp
from jax import lax
from jax.experimental import pallas as pl
from jax.experimental.pallas import tpu as pltpu
from jax.experimental.pallas import tpu_sc as plsc

N = 10000
D = 128
E = 320000

NC = 2
NS = 16
NW = NC * NS
K = 128
CH = 79
EPW = K * CH
EPAD = EPW * NW
NPAD = 10240
ZROWS = NPAD // NS
OROWS = 624
DW = 16

_mesh = plsc.VectorSubcoreMesh(core_axis_name="c", subcore_axis_name="s")


def _deg_body(dst_hbm, out_hbm, dst_v, buf_v, acc_sh):
    c = lax.axis_index("c")
    s = lax.axis_index("s")
    wid = s * NC + c

    def _zrow(i, carry):
        buf_v[i] = jnp.zeros((DW,), jnp.float32)
        return carry

    lax.fori_loop(0, K, _zrow, 0)
    for m in range(ZROWS // K):
        pltpu.sync_copy(buf_v, acc_sh.at[pl.ds(s * ZROWS + m * K, K)])

    def _orow(i, carry):
        buf_v[i] = jnp.ones((DW,), jnp.float32)
        return carry

    lax.fori_loop(0, K, _orow, 0)
    plsc.subcore_barrier()

    def _body(j, carry):
        off = wid * EPW + j * K
        pltpu.sync_copy(dst_hbm.at[pl.ds(off, K)], dst_v)
        pltpu.sync_copy(buf_v, acc_sh.at[dst_v], add=True)
        return carry

    lax.fori_loop(0, CH, _body, 0)
    plsc.subcore_barrier()

    pltpu.sync_copy(acc_sh.at[pl.ds(s * OROWS, OROWS)],
                    out_hbm.at[c, pl.ds(s * OROWS, OROWS)])

    @pl.when(s == NS - 1)
    def _tail():
        pltpu.sync_copy(acc_sh.at[pl.ds(NS * OROWS, N - NS * OROWS)],
                        out_hbm.at[c, pl.ds(NS * OROWS, N - NS * OROWS)])


_deg_kernel = pl.kernel(
    _deg_body,
    out_type=jax.ShapeDtypeStruct((NC, N, DW), jnp.float32),
    mesh=_mesh,
    scratch_types=[
        pltpu.VMEM((K,), jnp.int32),
        pltpu.VMEM((K, DW), jnp.float32),
        pltpu.VMEM_SHARED((NPAD, DW), jnp.float32),
    ],
)


def _edge_body(g_hbm, src_hbm, dst_hbm, out_hbm, src_v, dst_v, rows_v, acc_sh,
               sem):
    c = lax.axis_index("c")
    s = lax.axis_index("s")
    wid = s * NC + c

    def _zrow(i, carry):
        for j in range(D // 16):
            rows_v[i, pl.ds(j * 16, 16)] = jnp.zeros((16,), jnp.float32)
        return carry

    lax.fori_loop(0, K, _zrow, 0)
    for m in range(ZROWS // K):
        pltpu.sync_copy(rows_v, acc_sh.at[pl.ds(s * ZROWS + m * K, K)])
    plsc.subcore_barrier()

    def _body(j, carry):
        off = wid * EPW + j * K
        pltpu.sync_copy(src_hbm.at[pl.ds(off, K)], src_v)
        pltpu.sync_copy(dst_hbm.at[pl.ds(off, K)], dst_v)
        pltpu.async_copy(g_hbm.at[src_v], rows_v, sem).wait()
        pltpu.sync_copy(rows_v, acc_sh.at[dst_v], add=True)
        return carry

    lax.fori_loop(0, CH, _body, 0)
    plsc.subcore_barrier()

    pltpu.sync_copy(acc_sh.at[pl.ds(s * OROWS, OROWS)],
                    out_hbm.at[c, pl.ds(s * OROWS, OROWS)])

    @pl.when(s == NS - 1)
    def _tail():
        pltpu.sync_copy(acc_sh.at[pl.ds(NS * OROWS, N - NS * OROWS)],
                        out_hbm.at[c, pl.ds(NS * OROWS, N - NS * OROWS)])


_edge_kernel = pl.kernel(
    _edge_body,
    out_type=jax.ShapeDtypeStruct((NC, N, D), jnp.float32),
    mesh=_mesh,
    scratch_types=[
        pltpu.VMEM((K,), jnp.int32),
        pltpu.VMEM((K,), jnp.int32),
        pltpu.VMEM((K, D), jnp.float32),
        pltpu.VMEM_SHARED((NPAD, D), jnp.float32),
        pltpu.SemaphoreType.DMA,
    ],
)


R = 1000


def _tc1_body(x_ref, w1_ref, p_ref, h_ref, g_ref, dinv_ref):
    p = p_ref[...]
    deg = 1.0 + p[0, :, 0:1] + p[1, :, 0:1]
    dinv = lax.rsqrt(deg)
    h = jnp.dot(x_ref[...], w1_ref[...], preferred_element_type=jnp.float32)
    h_ref[...] = h
    g_ref[...] = h * dinv
    dinv_ref[...] = dinv


def _tc1(x, W1, p):
    return pl.pallas_call(
        _tc1_body,
        grid=(N // R,),
        in_specs=[
            pl.BlockSpec((R, D), lambda i: (i, 0)),
            pl.BlockSpec((D, D), lambda i: (0, 0)),
            pl.BlockSpec((NC, R, DW), lambda i: (0, i, 0)),
        ],
        out_specs=[
            pl.BlockSpec((R, D), lambda i: (i, 0)),
            pl.BlockSpec((R, D), lambda i: (i, 0)),
            pl.BlockSpec((R, 1), lambda i: (i, 0)),
        ],
        out_shape=[
            jax.ShapeDtypeStruct((N, D), jnp.float32),
            jax.ShapeDtypeStruct((N, D), jnp.float32),
            jax.ShapeDtypeStruct((N, 1), jnp.float32),
        ],
    )(x, W1, p)


def _tc2_body(acc_ref, h1_ref, dinv_ref, b1_ref, w2_ref, h2_ref, g2_ref):
    dinv = dinv_ref[...]
    a = acc_ref[0] + acc_ref[1]
    z = jnp.maximum(a * dinv + h1_ref[...] * (dinv * dinv) + b1_ref[...], 0.0)
    h2 = jnp.dot(z, w2_ref[...], preferred_element_type=jnp.float32)
    h2_ref[...] = h2
    g2_ref[...] = h2 * dinv


def _tc2(acc, h1, dinv, b1, W2):
    return pl.pallas_call(
        _tc2_body,
        grid=(N // R,),
        in_specs=[
            pl.BlockSpec((NC, R, D), lambda i: (0, i, 0)),
            pl.BlockSpec((R, D), lambda i: (i, 0)),
            pl.BlockSpec((R, 1), lambda i: (i, 0)),
            pl.BlockSpec((1, D), lambda i: (0, 0)),
            pl.BlockSpec((D, D), lambda i: (0, 0)),
        ],
        out_specs=[
            pl.BlockSpec((R, D), lambda i: (i, 0)),
            pl.BlockSpec((R, D), lambda i: (i, 0)),
        ],
        out_shape=[
            jax.ShapeDtypeStruct((N, D), jnp.float32),
            jax.ShapeDtypeStruct((N, D), jnp.float32),
        ],
    )(acc, h1, dinv, b1, W2)


def _tc3_body(acc_ref, h2_ref, dinv_ref, b2_ref, fw1_ref, fb1_ref, fw2_ref,
              fb2_ref, out_ref):
    dinv = dinv_ref[...]
    a = acc_ref[0] + acc_ref[1]
    z = jnp.maximum(a * dinv + h2_ref[...] * (dinv * dinv) + b2_ref[...], 0.0)
    t = jnp.maximum(
        jnp.dot(z, fw1_ref[...], preferred_element_type=jnp.float32)
        + fb1_ref[...], 0.0)
    out_ref[...] = (
        jnp.dot(t, fw2_ref[...], preferred_element_type=jnp.float32)
        + fb2_ref[...])


def _tc3(acc, h2, dinv, b2, fcW1, fcb1, fcW2, fcb2):
    return pl.pallas_call(
        _tc3_body,
        grid=(N // R,),
        in_specs=[
            pl.BlockSpec((NC, R, D), lambda i: (0, i, 0)),
            pl.BlockSpec((R, D), lambda i: (i, 0)),
            pl.BlockSpec((R, 1), lambda i: (i, 0)),
            pl.BlockSpec((1, D), lambda i: (0, 0)),
            pl.BlockSpec((D, D), lambda i: (0, 0)),
            pl.BlockSpec((1, D), lambda i: (0, 0)),
            pl.BlockSpec((D, D), lambda i: (0, 0)),
            pl.BlockSpec((1, D), lambda i: (0, 0)),
        ],
        out_specs=pl.BlockSpec((R, D), lambda i: (i, 0)),
        out_shape=jax.ShapeDtypeStruct((N, D), jnp.float32),
    )(acc, h2, dinv, b2, fcW1, fcb1, fcW2, fcb2)


def kernel(x, edge_index, W1, b1, W2, b2, fcW1, fcb1, fcW2, fcb2):
    src = edge_index[0]
    dst = edge_index[1]
    pad = EPAD - E
    src_p = jnp.concatenate([src, jnp.zeros((pad,), jnp.int32)])
    dst_p = jnp.concatenate([dst, jnp.full((pad,), N, jnp.int32)])

    p = _deg_kernel(dst_p)
    h1, g1, dinv = _tc1(x, W1, p)
    acc1 = _edge_kernel(g1, src_p, dst_p)
    h2, g2 = _tc2(acc1, h1, dinv, b1.reshape(1, D), W2)
    acc2 = _edge_kernel(g2, src_p, dst_p)
    return _tc3(acc2, h2, dinv, b2.reshape(1, D), fcW1, fcb1.reshape(1, D),
                fcW2, fcb2.reshape(1, D))
